# baseline (speedup 1.0000x reference)
import jax
import jax.numpy as jnp
from jax import lax
from jax.experimental import pallas as pl
from jax.experimental.pallas import tpu as pltpu
from jax.experimental.pallas import tpu_sc as plsc

NUM_ITEMS = 1000000
SIDE_VOCAB = 100000
N_SIDE = 2
EMB = 64
B = 16384
NCTX = 5

NC = 2
NS = 16
L = 16
NW = NC * NS
BW = B // NW
CH = 64
NCHUNK = BW // CH
NBUF = 2


def _softmax3(w0, w1, w2):
    m = jnp.maximum(w0, jnp.maximum(w1, w2))
    e0 = jnp.exp(w0 - m)
    e1 = jnp.exp(w1 - m)
    e2 = jnp.exp(w2 - m)
    s = e0 + e1 + e2
    return e0 / s, e1 / s, e2 / s


def _body(ci_hbm, csi_hbm, ctx_hbm, ein_hbm, eout_hbm, wt_hbm, side_hbm,
          out_hbm, *scratch):
    per = 12
    slots = [scratch[i * per:(i + 1) * per] for i in range(NBUF)]

    wid = lax.axis_index("s") * NC + lax.axis_index("c")
    base0 = wid * BW

    iota16 = lax.iota(jnp.int32, L)

    def issue(k):
        ii, is0, is1, ic, iw, ri, rs0, rs1, rc, w3, ob, sem = slots[k % NBUF]
        base = base0 + k * CH
        pltpu.sync_copy(ci_hbm.at[pl.ds(base, CH)], ii)
        pltpu.sync_copy(csi_hbm.at[pl.ds(base, CH)], is0)
        pltpu.sync_copy(csi_hbm.at[pl.ds(B + base, CH)], is1)
        pltpu.sync_copy(ctx_hbm.at[pl.ds(base * NCTX, CH * NCTX)], ic)
        for t in range(CH // L):
            v = ii[pl.ds(t * L, L)] * 3
            iw[pl.ds(t * L, L)] = v
            iw[pl.ds(CH + t * L, L)] = v + 1
            iw[pl.ds(2 * CH + t * L, L)] = v + 2
        hs = [
            pltpu.make_async_copy(ein_hbm.at[ii], ri, sem),
            pltpu.make_async_copy(side_hbm.at[is0], rs0, sem),
            pltpu.make_async_copy(side_hbm.at[is1], rs1, sem),
        ]
        for j in range(3):
            hs.append(pltpu.make_async_copy(
                wt_hbm.at[iw.at[pl.ds(j * CH, CH)]],
                w3.at[pl.ds(j * CH, CH)], sem))
        for off, ln in ((0, 128), (128, 128), (256, 64)):
            hs.append(pltpu.make_async_copy(
                eout_hbm.at[ic.at[pl.ds(off, ln)]],
                rc.at[pl.ds(off, ln)], sem))
        for h in hs:
            h.start()
        return hs

    def compute(k):
        ii, is0, is1, ic, iw, ri, rs0, rs1, rc, w3, ob, sem = slots[k % NBUF]
        zf = jnp.zeros((L,), jnp.float32)

        def group(g, carry):
            o = g * L
            lane = jnp.full((L,), o, jnp.int32) + iota16
            lane5 = lane * NCTX
            w0 = w3[pl.ds(o, L)]
            w1 = w3[pl.ds(CH + o, L)]
            w2 = w3[pl.ds(2 * CH + o, L)]
            p0, p1, p2 = _softmax3(w0, w1, w2)

            def dbody(d, accs):
                dv = jnp.full((L,), d, jnp.int32)
                h = (p0 * plsc.load_gather(ri, [lane, dv])
                     + p1 * plsc.load_gather(rs0, [lane, dv])
                     + p2 * plsc.load_gather(rs1, [lane, dv]))
                return tuple(
                    accs[c] + h * plsc.load_gather(
                        rc, [lane5 + jnp.full((L,), c, jnp.int32), dv])
                    for c in range(NCTX))

            accs = lax.fori_loop(0, EMB, dbody, (zf,) * NCTX)
            for c in range(NCTX):
                sig = 1.0 / (1.0 + jnp.exp(-accs[c]))
                plsc.store_scatter(
                    ob, [lane5 + jnp.full((L,), c, jnp.int32)], sig)
            return carry

        lax.fori_loop(0, CH // L, group, 0)

    pending = issue(0)
    for k in range(NCHUNK):
        nxt = issue(k + 1) if k + 1 < NCHUNK else None
        for h in pending:
            h.wait()
        compute(k)
        ob = slots[k % NBUF][10]
        pltpu.sync_copy(
            ob, out_hbm.at[pl.ds((base0 + k * CH) * NCTX, CH * NCTX)])
        pending = nxt


def _scratch_types():
    per_slot = [
        pltpu.VMEM((CH,), jnp.int32),
        pltpu.VMEM((CH,), jnp.int32),
        pltpu.VMEM((CH,), jnp.int32),
        pltpu.VMEM((CH * NCTX,), jnp.int32),
        pltpu.VMEM((CH * 3,), jnp.int32),
        pltpu.VMEM((CH, EMB), jnp.float32),
        pltpu.VMEM((CH, EMB), jnp.float32),
        pltpu.VMEM((CH, EMB), jnp.float32),
        pltpu.VMEM((CH * NCTX, EMB), jnp.float32),
        pltpu.VMEM((CH * 3,), jnp.float32),
        pltpu.VMEM((CH * NCTX,), jnp.float32),
        pltpu.SemaphoreType.DMA,
    ]
    return per_slot * NBUF


@jax.jit
def kernel(central_items, central_side_informations, context_items,
           item_embedding_in, item_embedding_out, weights_table, side_tables):
    ci = central_items.astype(jnp.int32)
    csi = (central_side_informations.astype(jnp.int32)
           + (jnp.arange(N_SIDE, dtype=jnp.int32) * SIDE_VOCAB)[:, None]
           ).reshape(-1)
    ctx = context_items.astype(jnp.int32).reshape(-1)
    side_flat = side_tables.reshape(N_SIDE * SIDE_VOCAB, EMB)
    wt_flat = weights_table.reshape(-1)

    mesh = plsc.VectorSubcoreMesh(
        core_axis_name="c", subcore_axis_name="s",
        num_cores=NC, num_subcores=NS)
    run = pl.kernel(
        _body,
        out_type=jax.ShapeDtypeStruct((B * NCTX,), jnp.float32),
        mesh=mesh,
        scratch_types=_scratch_types(),
        compiler_params=pltpu.CompilerParams(
            needs_layout_passes=False, use_tc_tiling_on_sc=False),
    )
    out = run(ci, csi, ctx, item_embedding_in, item_embedding_out,
              wt_flat, side_flat)
    return out.reshape(B, NCTX)
